# baseline (speedup 1.0000x reference)
import jax
import jax.numpy as jnp
from jax.experimental import pallas as pl
from jax.experimental.pallas import tpu as pltpu

_BATCH = 4096
_DIM = 128
_NCHUNK = 2
_ROWS = _BATCH // _NCHUNK


def _copy_body(x_ref, o_ref, buf, in_sem, out_sem):
    def in_copy(i):
        return pltpu.make_async_copy(
            x_ref.at[pl.ds(i * _ROWS, _ROWS)], buf.at[i], in_sem)

    def out_copy(i):
        return pltpu.make_async_copy(
            buf.at[i], o_ref.at[pl.ds(i * _ROWS, _ROWS)], out_sem)

    in_copy(0).start()
    in_copy(1).start()
    in_copy(0).wait()
    out_copy(0).start()
    in_copy(1).wait()
    out_copy(1).start()
    out_copy(0).wait()
    out_copy(1).wait()


def kernel(initial_space, finite_space, time_embedding):
    return pl.pallas_call(
        _copy_body,
        in_specs=[pl.BlockSpec(memory_space=pl.ANY)],
        out_specs=pl.BlockSpec(memory_space=pl.ANY),
        out_shape=jax.ShapeDtypeStruct((_BATCH, _DIM), jnp.float32),
        scratch_shapes=[
            pltpu.VMEM((_NCHUNK, _ROWS, _DIM), jnp.float32),
            pltpu.SemaphoreType.DMA,
            pltpu.SemaphoreType.DMA,
        ],
    )(initial_space)


# speedup vs baseline: 1.0398x; 1.0398x over previous
import jax
import jax.numpy as jnp
from jax.experimental import pallas as pl
from jax.experimental.pallas import tpu as pltpu

_BATCH = 4096
_DIM = 128
_CHUNKS = (1360, 1368, 1368)
_OFFS = tuple(sum(_CHUNKS[:i]) for i in range(len(_CHUNKS)))
_NCHUNK = len(_CHUNKS)
_MAXROWS = max(_CHUNKS)


def _copy_body(x_ref, o_ref, buf, in_sem, out_sem):
    def in_copy(i):
        return pltpu.make_async_copy(
            x_ref.at[pl.ds(_OFFS[i], _CHUNKS[i])],
            buf.at[i, pl.ds(0, _CHUNKS[i])], in_sem)

    def out_copy(i):
        return pltpu.make_async_copy(
            buf.at[i, pl.ds(0, _CHUNKS[i])],
            o_ref.at[pl.ds(_OFFS[i], _CHUNKS[i])], out_sem)

    for i in range(_NCHUNK):
        in_copy(i).start()
    for i in range(_NCHUNK):
        in_copy(i).wait()
        out_copy(i).start()
    for i in range(_NCHUNK):
        out_copy(i).wait()


def kernel(initial_space, finite_space, time_embedding):
    return pl.pallas_call(
        _copy_body,
        in_specs=[pl.BlockSpec(memory_space=pl.ANY)],
        out_specs=pl.BlockSpec(memory_space=pl.ANY),
        out_shape=jax.ShapeDtypeStruct((_BATCH, _DIM), jnp.float32),
        scratch_shapes=[
            pltpu.VMEM((_NCHUNK, _MAXROWS, _DIM), jnp.float32),
            pltpu.SemaphoreType.DMA,
            pltpu.SemaphoreType.DMA,
        ],
    )(initial_space)
